# baseline (speedup 1.0000x reference)
import jax
import jax.numpy as jnp
from jax import lax
from jax.experimental import pallas as pl
from jax.experimental.pallas import tpu as pltpu
from jax.experimental.pallas import tpu_sc as plsc

NUM_BUCKETS = 32
NUM_HEADS = 12
QL = 2048
KL = 2048

NC = 2
NS = 16
LANES = 16
NW = NC * NS
NSHIFT = 8
BAND_LO = 1664
BAND_W = 768
BAND_STEPS = BAND_W // LANES
N_UNITS = NUM_HEADS * NSHIFT

_THRESHOLDS = (12, 16, 23, 32, 46, 64, 91)


def _bucket_of(d):
  rel = d - (QL - 1)
  n = jnp.abs(rel)
  large = jnp.full((LANES,), 8, dtype=jnp.int32)
  for thr in _THRESHOLDS:
    large = large + jnp.where(n >= thr, 1, 0).astype(jnp.int32)
  return jnp.where(n < 8, n, large) + jnp.where(rel > 0, 16, 0)


def _band_body(bias_hbm, r8_hbm, bias_v, buf_v):
  wid = lax.axis_index("s") * NC + lax.axis_index("c")
  pltpu.sync_copy(bias_hbm, bias_v)

  for j in range(N_UNITS // NW):
    u = wid + NW * j
    h = lax.shift_right_logical(u, 3)
    k = lax.bitwise_and(u, NSHIFT - 1)
    head_idx = jnp.full((LANES,), h, dtype=jnp.int32)

    def step(t, carry, k=k, head_idx=head_idx):
      d = BAND_LO + 7 - k + t * LANES + lax.iota(jnp.int32, LANES)
      vals = plsc.load_gather(bias_v, [_bucket_of(d), head_idx])
      buf_v[pl.ds(t * LANES, LANES)] = vals
      return carry

    lax.fori_loop(0, BAND_STEPS, step, 0)
    pltpu.sync_copy(buf_v, r8_hbm.at[h, k])


_sc_band = pl.kernel(
    _band_body,
    out_type=jax.ShapeDtypeStruct((NUM_HEADS, NSHIFT, BAND_W), jnp.float32),
    mesh=plsc.VectorSubcoreMesh(core_axis_name="c", subcore_axis_name="s"),
    compiler_params=pltpu.CompilerParams(
        needs_layout_passes=False, use_tc_tiling_on_sc=False
    ),
    scratch_types=[
        pltpu.VMEM((NUM_BUCKETS, NUM_HEADS), jnp.float32),
        pltpu.VMEM((BAND_W,), jnp.float32),
    ],
)

_BROWS = 128
_GRID_G = QL // _BROWS


def _tc_body(bias_s, r8_ref, out_ref):
  h = pl.program_id(0)
  g = pl.program_id(1)
  cn = bias_s[15, h]
  cp = bias_s[31, h]
  for ct in range(16):
    cval = jnp.where(ct < g, cn, cp)
    out_ref[0, :, 128 * ct:128 * (ct + 1)] = jnp.full(
        (_BROWS, 128), cval, jnp.float32
    )
  for dd in (-1, 0, 1):
    cond = jnp.logical_and(g + dd >= 0, g + dd <= _GRID_G - 1)

    @pl.when(cond)
    def _(dd=dd):
      pieces = [
          r8_ref[0, :, 376 + 128 * dd - 8 * a:504 + 128 * dd - 8 * a]
          for a in range(16)
      ]
      w = jnp.concatenate(pieces, axis=0)
      out_ref[0, :, pl.ds((g + dd) * 128, 128)] = w


_tc_fill = pl.pallas_call(
    _tc_body,
    grid=(NUM_HEADS, _GRID_G),
    in_specs=[
        pl.BlockSpec(memory_space=pltpu.SMEM),
        pl.BlockSpec((1, NSHIFT, BAND_W), lambda h, g: (h, 0, 0)),
    ],
    out_specs=pl.BlockSpec((1, _BROWS, KL), lambda h, g: (h, g, 0)),
    out_shape=jax.ShapeDtypeStruct((NUM_HEADS, QL, KL), jnp.float32),
    compiler_params=pltpu.CompilerParams(
        dimension_semantics=("parallel", "parallel")
    ),
)


@jax.jit
def kernel(query_length, key_length, relative_attention_bias):
  del query_length, key_length
  r8 = _sc_band(relative_attention_bias)
  return _tc_fill(relative_attention_bias, r8)
